# baseline (speedup 1.0000x reference)
import functools

import jax
import jax.numpy as jnp
from jax import lax
from jax.experimental import pallas as pl
from jax.experimental.pallas import tpu as pltpu
from jax.experimental.pallas import tpu_sc as plsc

B = 16384
L = 200
E = 18
EP = 32
G = 16
NC = 2
NS = 16
NW = NC * NS
BPW = B // NW
POS = BPW * L
NP = 4
NPP = BPW // NP
POSH = POS // NP
CH = 512
NCH = POSH // CH
NSL = 2

_mesh = plsc.VectorSubcoreMesh(
    core_axis_name="c", subcore_axis_name="s", num_cores=NC, num_subcores=NS)

_f32 = jnp.float32
_i32 = jnp.int32


@functools.partial(
    pl.kernel,
    out_type=[
        pltpu.HBM((2 * B, G), _f32),
        pltpu.HBM((2 * B, G), _f32),
        pltpu.HBM((B, EP), _f32),
        pltpu.HBM((16 * B, G), _f32),
        pltpu.HBM((B, EP), _f32),
    ],
    mesh=_mesh,
    scratch_types=[
        [pltpu.VMEM((2 * CH,), _i32)] * NSL,
        [pltpu.VMEM((2 * CH,), _i32)] * NSL,
        [pltpu.VMEM((2 * CH, G), _f32)] * NSL,
        [pltpu.VMEM((CH,), _i32)] * NSL,
        [pltpu.VMEM((CH,), _i32)] * NSL,
        [pltpu.VMEM((CH, EP), _f32)] * NSL,
        [pltpu.SemaphoreType.DMA] * NSL,
        [pltpu.SemaphoreType.DMA] * NSL,
        [pltpu.SemaphoreType.DMA] * NSL,
        pltpu.VMEM_SHARED((NS * NPP * 16, G), _f32),
        pltpu.VMEM_SHARED((NS * NPP, EP), _f32),
    ],
    compiler_params=pltpu.CompilerParams(use_tc_tiling_on_sc=False),
)
def _sc_embed(uidg, midg, cat_i, mhg, mha, ch_i, lmap, z3, zc,
              uid_t, mid_t, cat_t,
              uid_o, mid_o, cat_o, mids_o, cats_o,
              gm_idx, am_idx, row_m, gc_idx, ac_idx, row_c,
              si, sg, ss, acc_m, acc_c):
  c = lax.axis_index("c")
  s = lax.axis_index("s")
  wid = s * NC + c
  base = wid * BPW
  arow = s * NPP
  arow3 = s * NPP * 16

  for pidx, table, out in ((uidg, uid_t, uid_o), (midg, mid_t, mid_o)):
    pltpu.sync_copy(pidx.at[pl.ds(2 * base, 2 * BPW)], gm_idx[0])
    pltpu.sync_copy(table.at[gm_idx[0]], row_m[0])
    pltpu.sync_copy(row_m[0], out.at[pl.ds(2 * base, 2 * BPW)])
  pltpu.sync_copy(cat_i.at[pl.ds(base, BPW)], gc_idx[0])
  pltpu.sync_copy(cat_t.at[gc_idx[0]], row_c[0])
  pltpu.sync_copy(row_c[0], cat_o.at[pl.ds(base, BPW)])

  posbase = wid * POS

  def _stage_idx(ci, off, b):
    pltpu.async_copy(mhg.at[pl.ds(2 * off, 2 * CH)], gm_idx[b], si[b])
    pltpu.async_copy(mha.at[pl.ds(2 * off, 2 * CH)], am_idx[b], si[b])
    pltpu.async_copy(ch_i.at[pl.ds(off, CH)], gc_idx[b], si[b])
    pltpu.async_copy(lmap.at[pl.ds(s * POSH + ci * CH, CH)], ac_idx[b], si[b])

  def _wait_idx(b):
    for dst in (gm_idx[b], am_idx[b]):
      pltpu.make_async_copy(mhg.at[pl.ds(0, 2 * CH)], dst, si[b]).wait()
    for dst in (gc_idx[b], ac_idx[b]):
      pltpu.make_async_copy(ch_i.at[pl.ds(0, CH)], dst, si[b]).wait()

  def _wait_scat(b):
    pltpu.make_async_copy(row_m[b], acc_m.at[am_idx[b]], ss[b]).wait()
    pltpu.make_async_copy(row_c[b], acc_c.at[ac_idx[b]], ss[b]).wait()

  for p in range(NP):
    pltpu.sync_copy(z3, acc_m.at[pl.ds(arow3, NPP * 16)])
    pltpu.sync_copy(zc, acc_c.at[pl.ds(arow, NPP)])
    pbase = posbase + p * POSH
    _stage_idx(0, pbase, 0)

    def _ring(gi, carry):
      for b in range(NSL):
        ci = gi * NSL + b
        nb = 1 - b

        _wait_idx(b)
        gm = pltpu.async_copy(mid_t.at[gm_idx[b]], row_m[b], sg[b])
        gc = pltpu.async_copy(cat_t.at[gc_idx[b]], row_c[b], sg[b])

        if b == 0:
          @pl.when(gi >= 1)
          def _():
            _wait_scat(nb)
        else:
          _wait_scat(nb)

        if b == NSL - 1:
          @pl.when(gi < NCH // NSL - 1)
          def _():
            _stage_idx(ci + 1, pbase + (ci + 1) * CH, nb)
        else:
          _stage_idx(ci + 1, pbase + (ci + 1) * CH, nb)

        gm.wait()
        gc.wait()
        pltpu.async_copy(row_m[b], acc_m.at[am_idx[b]], ss[b], add=True)
        pltpu.async_copy(row_c[b], acc_c.at[ac_idx[b]], ss[b], add=True)
      return carry
    lax.fori_loop(0, NCH // NSL, _ring, 0)
    _wait_scat(NSL - 1)

    pltpu.sync_copy(acc_m.at[pl.ds(arow3, NPP * 16)],
                    mids_o.at[pl.ds((base + p * NPP) * 16, NPP * 16)])
    pltpu.sync_copy(acc_c.at[pl.ds(arow, NPP)],
                    cats_o.at[pl.ds(base + p * NPP, NPP)])


MB = 512


def _fix_single(raw, ph):
  out = jnp.zeros((MB, E), _f32)
  for k in range(8):
    out = out + jnp.where(ph == k, raw[:, 2 * k:2 * k + E], 0.0)
  return out


def _mlp_body(uraw, uph, mraw, mph, ce, msraw, cs, w1, b1, w2, b2, w3, b3, o):
  u = _fix_single(uraw[...], uph[...])
  m = _fix_single(mraw[...], mph[...])
  ms = jnp.zeros((MB, E), _f32)
  for k in range(8):
    ms = ms + msraw[:, 34 * k:34 * k + E]
  inp = jnp.concatenate([u, m, ce[:, :E], ms, cs[:, :E]], axis=1)
  h = jnp.dot(inp, w1[...], preferred_element_type=_f32) + b1[...]
  h = jnp.maximum(h, 0.0)
  h = jnp.dot(h, w2[...], preferred_element_type=_f32) + b2[...]
  h = jnp.maximum(h, 0.0)
  o[...] = jnp.dot(h, w3[...], preferred_element_type=_f32) + b3[...]


def _mlp(uraw, uph, mraw, mph, ce, msraw, cs, w1, b1, w2, b2, w3, b3):
  blk = lambda w: pl.BlockSpec((MB, w), lambda i: (i, 0))
  full = lambda a: pl.BlockSpec(a.shape, lambda i: (0,) * a.ndim)
  return pl.pallas_call(
      _mlp_body,
      grid=(B // MB,),
      in_specs=[blk(2 * G), blk(1), blk(2 * G), blk(1), blk(EP), blk(16 * G),
                blk(EP), full(w1), full(b1), full(w2), full(b2), full(w3),
                full(b3)],
      out_specs=pl.BlockSpec((MB, 1), lambda i: (i, 0)),
      out_shape=jax.ShapeDtypeStruct((B, 1), _f32),
  )(uraw, uph, mraw, mph, ce, msraw, cs, w1, b1, w2, b2, w3, b3)


def _pairs(r):
  g = (9 * r) // 8
  return jnp.stack([g, g + 1], axis=-1).reshape(-1)


def kernel(uid_batch_ph, mid_batch_ph, mid_his_batch_ph, cat_batch_ph,
           cat_his_batch_ph, mask, seq_len_ph, target_ph, lr,
           uid_table, mid_table, cat_table, W1, b1, W2, b2, W3, b3):
  uid_i = uid_batch_ph.astype(_i32)
  mid_i = mid_batch_ph.astype(_i32)
  cat_i = cat_batch_ph.astype(_i32)
  mh_i = mid_his_batch_ph.astype(_i32).reshape(B * L)
  ch_i = cat_his_batch_ph.astype(_i32).reshape(B * L)

  uidg = _pairs(uid_i)
  midg = _pairs(mid_i)
  mhg = _pairs(mh_i)

  p_grid = jnp.arange(B * L, dtype=_i32)
  s_arr = (p_grid // POS) // NC
  lb = (p_grid // L) % NPP
  d = (s_arr * NPP + lb) * 16 + 2 * (mh_i % 8)
  mha = jnp.stack([d, d + 1], axis=-1).reshape(-1)

  lmap = (jnp.arange(POSH, dtype=_i32) // L)[None, :] \
      + (jnp.arange(NS, dtype=_i32) * NPP)[:, None]
  lmap = lmap.reshape(NS * POSH)

  z3 = jnp.zeros((NPP * 16, G), _f32)
  zc = jnp.zeros((NPP, EP), _f32)

  uid_raw, mid_raw, cat_e, mids_raw, cats = _sc_embed(
      uidg, midg, cat_i, mhg, mha, ch_i, lmap, z3, zc,
      uid_table.astype(_f32).reshape(1125000, G),
      mid_table.astype(_f32).reshape(1125000, G),
      jnp.pad(cat_table.astype(_f32), ((0, 0), (0, EP - E))))

  return _mlp(uid_raw.reshape(B, 2 * G), (uid_i % 8).reshape(B, 1),
              mid_raw.reshape(B, 2 * G), (mid_i % 8).reshape(B, 1),
              cat_e, mids_raw.reshape(B, 16 * G), cats,
              W1, b1.reshape(1, 200), W2, b2.reshape(1, 80),
              W3, b3.reshape(1, 1))


# speedup vs baseline: 2.3399x; 2.3399x over previous
import functools

import jax
import jax.numpy as jnp
from jax import lax
from jax.experimental import pallas as pl
from jax.experimental.pallas import tpu as pltpu
from jax.experimental.pallas import tpu_sc as plsc

B = 16384
L = 200
E = 18
EP = 32
NC = 2
NS = 16
NW = NC * NS
BPW = B // NW
POS = BPW * L
NP = 2
NPP = BPW // NP
POSH = POS // NP
CH = 512
NCH = POSH // CH
NSL = 2

_mesh = plsc.VectorSubcoreMesh(
    core_axis_name="c", subcore_axis_name="s", num_cores=NC, num_subcores=NS)

_f32 = jnp.float32
_i32 = jnp.int32


@functools.partial(
    pl.kernel,
    out_type=[pltpu.HBM((B, EP), _f32)] * 5,
    mesh=_mesh,
    scratch_types=[
        [pltpu.VMEM((CH,), _i32)] * NSL,
        [pltpu.VMEM((CH,), _i32)] * NSL,
        [pltpu.VMEM((CH, EP), _f32)] * NSL,
        [pltpu.VMEM((CH, EP), _f32)] * NSL,
        [pltpu.VMEM((CH,), _i32)] * NSL,
        pltpu.VMEM((BPW,), _i32),
        pltpu.VMEM((BPW, EP), _f32),
        [pltpu.SemaphoreType.DMA] * NSL,
        [pltpu.SemaphoreType.DMA] * NSL,
        [pltpu.SemaphoreType.DMA] * NSL,
        pltpu.VMEM_SHARED((NS * NPP, EP), _f32),
        pltpu.VMEM_SHARED((NS * NPP, EP), _f32),
    ],
    compiler_params=pltpu.CompilerParams(use_tc_tiling_on_sc=False),
)
def _sc_embed(uid_i, mid_i, cat_i, mh_i, ch_i, lmap, zrows, uid_t, mid_t,
              cat_t, uid_o, mid_o, cat_o, mids_o, cats_o,
              idx_m, idx_c, row_m, row_c, b_idx, sidx, srow,
              si, sg, ss, acc_m, acc_c):
  c = lax.axis_index("c")
  s = lax.axis_index("s")
  wid = s * NC + c
  base = wid * BPW
  arow = s * NPP

  for idx1d, table, out in ((uid_i, uid_t, uid_o),
                            (mid_i, mid_t, mid_o),
                            (cat_i, cat_t, cat_o)):
    pltpu.sync_copy(idx1d.at[pl.ds(base, BPW)], sidx)
    pltpu.sync_copy(table.at[sidx], srow)
    pltpu.sync_copy(srow, out.at[pl.ds(base, BPW)])

  posbase = wid * POS

  def _stage_idx(ci, off, b):
    pltpu.async_copy(mh_i.at[pl.ds(off, CH)], idx_m[b], si[b])
    pltpu.async_copy(ch_i.at[pl.ds(off, CH)], idx_c[b], si[b])
    pltpu.async_copy(lmap.at[pl.ds(s * POSH + ci * CH, CH)], b_idx[b], si[b])

  def _wait_idx(b):
    for dst in (idx_m[b], idx_c[b], b_idx[b]):
      pltpu.make_async_copy(mh_i.at[pl.ds(0, CH)], dst, si[b]).wait()

  def _wait_scat(b):
    pltpu.make_async_copy(row_m[b], acc_m.at[b_idx[b]], ss[b]).wait()
    pltpu.make_async_copy(row_c[b], acc_c.at[b_idx[b]], ss[b]).wait()

  for p in range(NP):
    pltpu.sync_copy(zrows, acc_m.at[pl.ds(arow, NPP)])
    pltpu.sync_copy(zrows, acc_c.at[pl.ds(arow, NPP)])
    pbase = posbase + p * POSH
    _stage_idx(0, pbase, 0)

    def _ring(gi, carry):
      for b in range(NSL):
        ci = gi * NSL + b
        nb = 1 - b

        _wait_idx(b)
        gm = pltpu.async_copy(mid_t.at[idx_m[b]], row_m[b], sg[b])
        gc = pltpu.async_copy(cat_t.at[idx_c[b]], row_c[b], sg[b])

        if b == 0:
          @pl.when(gi >= 1)
          def _():
            _wait_scat(nb)
        else:
          _wait_scat(nb)

        if b == NSL - 1:
          @pl.when(gi < NCH // NSL - 1)
          def _():
            _stage_idx(ci + 1, pbase + (ci + 1) * CH, nb)
        else:
          _stage_idx(ci + 1, pbase + (ci + 1) * CH, nb)

        gm.wait()
        gc.wait()
        pltpu.async_copy(row_m[b], acc_m.at[b_idx[b]], ss[b], add=True)
        pltpu.async_copy(row_c[b], acc_c.at[b_idx[b]], ss[b], add=True)
      return carry
    lax.fori_loop(0, NCH // NSL, _ring, 0)
    _wait_scat(NSL - 1)

    pltpu.sync_copy(acc_m.at[pl.ds(arow, NPP)],
                    mids_o.at[pl.ds(base + p * NPP, NPP)])
    pltpu.sync_copy(acc_c.at[pl.ds(arow, NPP)],
                    cats_o.at[pl.ds(base + p * NPP, NPP)])


MB = 512


def _mlp_body(u, m, c, ms, cs, w1, b1, w2, b2, w3, b3, o):
  inp = jnp.concatenate([u[...], m[...], c[...], ms[...], cs[...]], axis=1)
  h = jnp.dot(inp, w1[...], preferred_element_type=_f32) + b1[...]
  h = jnp.maximum(h, 0.0)
  h = jnp.dot(h, w2[...], preferred_element_type=_f32) + b2[...]
  h = jnp.maximum(h, 0.0)
  o[...] = jnp.dot(h, w3[...], preferred_element_type=_f32) + b3[...]


def _mlp(u, m, c, ms, cs, w1p, b1, w2, b2, w3, b3):
  piece = pl.BlockSpec((MB, EP), lambda i: (i, 0))
  full = lambda a: pl.BlockSpec(a.shape, lambda i: (0,) * a.ndim)
  return pl.pallas_call(
      _mlp_body,
      grid=(B // MB,),
      in_specs=[piece] * 5 + [full(w1p), full(b1), full(w2), full(b2),
                              full(w3), full(b3)],
      out_specs=pl.BlockSpec((MB, 1), lambda i: (i, 0)),
      out_shape=jax.ShapeDtypeStruct((B, 1), _f32),
  )(u, m, c, ms, cs, w1p, b1, w2, b2, w3, b3)


def kernel(uid_batch_ph, mid_batch_ph, mid_his_batch_ph, cat_batch_ph,
           cat_his_batch_ph, mask, seq_len_ph, target_ph, lr,
           uid_table, mid_table, cat_table, W1, b1, W2, b2, W3, b3):
  uid_i = uid_batch_ph.astype(_i32)
  mid_i = mid_batch_ph.astype(_i32)
  cat_i = cat_batch_ph.astype(_i32)
  mh_i = mid_his_batch_ph.astype(_i32).reshape(B * L)
  ch_i = cat_his_batch_ph.astype(_i32).reshape(B * L)
  lmap = (jnp.arange(POSH, dtype=_i32) // L)[None, :] \
      + (jnp.arange(NS, dtype=_i32) * NPP)[:, None]
  lmap = lmap.reshape(NS * POSH)
  zrows = jnp.zeros((NPP, EP), _f32)

  pad = ((0, 0), (0, EP - E))
  uid_e, mid_e, cat_e, mids, cats = _sc_embed(
      uid_i, mid_i, cat_i, mh_i, ch_i, lmap, zrows,
      jnp.pad(uid_table.astype(_f32), pad),
      jnp.pad(mid_table.astype(_f32), pad),
      jnp.pad(cat_table.astype(_f32), pad))

  w1p = jnp.zeros((5 * EP, 200), _f32)
  for k in range(5):
    w1p = w1p.at[k * EP:k * EP + E].set(W1[k * E:(k + 1) * E])

  return _mlp(uid_e, mid_e, cat_e, mids, cats,
              w1p, b1.reshape(1, 200), W2, b2.reshape(1, 80),
              W3, b3.reshape(1, 1))
